# baseline (speedup 1.0000x reference)
import functools

import jax
import jax.numpy as jnp
from jax import lax
from jax.experimental import pallas as pl
from jax.experimental.pallas import tpu as pltpu
from jax.experimental.pallas import tpu_sc as plsc

N = 10000
D = 128
E = 320000
NW = 32
CHUNK = 128
CPT = 80
E_PAD = NW * CPT * CHUNK
N_ACC = 10240
GROUP = 16
K0 = 144
K1 = 2 * CPT - K0

_mesh = plsc.VectorSubcoreMesh(core_axis_name="c", subcore_axis_name="s")


_NP128 = N_ACC // 128


@functools.partial(
    pl.kernel,
    mesh=_mesh,
    out_type=jax.ShapeDtypeStruct((2 * N_ACC,), jnp.float32),
    scratch_types=[
        pltpu.VMEM((CPT, CHUNK), jnp.int32),
        pltpu.VMEM((CHUNK,), jnp.float32),
        pltpu.VMEM_SHARED((N_ACC,), jnp.float32),
    ],
)
def _deg_call(dst_hbm, zeros_hbm, ones_hbm, out_hbm, idx_v, ones_v, deg_sh):
    cid = lax.axis_index("c")
    sid = lax.axis_index("s")
    wid = cid * 16 + sid

    for k in range(5):
        p = sid + k * 16

        @pl.when(p < _NP128)
        def _():
            pltpu.sync_copy(zeros_hbm.at[pl.ds(p * 128, 128)],
                            deg_sh.at[pl.ds(p * 128, 128)])

    pltpu.sync_copy(ones_hbm, ones_v)
    pltpu.sync_copy(dst_hbm.at[pl.ds(wid * CPT, CPT)], idx_v)
    plsc.subcore_barrier()

    def chunk(j, carry):
        pltpu.sync_copy(ones_v, deg_sh.at[idx_v.at[j]], add=True)
        return carry

    lax.fori_loop(0, CPT, chunk, 0)
    plsc.subcore_barrier()
    for k in range(5):
        p = sid + k * 16

        @pl.when(p < _NP128)
        def _():
            pltpu.sync_copy(deg_sh.at[pl.ds(p * 128, 128)],
                            out_hbm.at[pl.ds(cid * N_ACC + p * 128, 128)])


@functools.partial(
    pl.kernel,
    mesh=_mesh,
    out_type=jax.ShapeDtypeStruct((2, N_ACC, D), jnp.float32),
    scratch_types=[
        pltpu.VMEM((GROUP, CHUNK), jnp.int32),
        pltpu.VMEM((GROUP, CHUNK), jnp.int32),
        pltpu.VMEM((CHUNK, D), jnp.float32),
        pltpu.VMEM((CHUNK, D), jnp.float32),
        pltpu.VMEM_SHARED((N_ACC, D), jnp.float32),
        pltpu.SemaphoreType.DMA,
        pltpu.SemaphoreType.DMA,
    ],
)
def _edge_call(g_hbm, src_hbm, dst_hbm, zeros_hbm, out_hbm,
               sidx_v, didx_v, rows0_v, rows1_v, acc_sh, sem0, sem1):
    cid = lax.axis_index("c")
    sid = lax.axis_index("s")
    pltpu.sync_copy(zeros_hbm.at[pl.ds(sid * 640, 640)],
                    acc_sh.at[pl.ds(sid * 640, 640)])
    plsc.subcore_barrier()

    ngroups = jnp.where(cid == 0, K0 // GROUP, K1 // GROUP)
    tile_base = sid * (2 * CPT) + jnp.where(cid == 0, 0, K0)

    def group(g, carry):
        base = pl.multiple_of(tile_base + g * GROUP, 8)
        pltpu.sync_copy(src_hbm.at[pl.ds(base, GROUP)], sidx_v)
        pltpu.sync_copy(dst_hbm.at[pl.ds(base, GROUP)], didx_v)
        pltpu.async_copy(g_hbm.at[cid].at[sidx_v.at[0]], rows0_v, sem0)

        def pair(t, carry2):
            j0 = 2 * t
            j1 = j0 + 1
            pltpu.async_copy(g_hbm.at[cid].at[sidx_v.at[j1]], rows1_v, sem1)
            pltpu.make_async_copy(g_hbm.at[cid].at[sidx_v.at[j0]], rows0_v,
                                  sem0).wait()
            pltpu.sync_copy(rows0_v, acc_sh.at[didx_v.at[j0]], add=True)

            @pl.when(j0 + 2 < GROUP)
            def _():
                pltpu.async_copy(g_hbm.at[cid].at[sidx_v.at[j0 + 2]], rows0_v, sem0)

            pltpu.make_async_copy(g_hbm.at[cid].at[sidx_v.at[j1]], rows1_v,
                                  sem1).wait()
            pltpu.sync_copy(rows1_v, acc_sh.at[didx_v.at[j1]], add=True)
            return carry2

        lax.fori_loop(0, GROUP // 2, pair, 0)
        return carry

    lax.fori_loop(0, ngroups, group, 0)
    plsc.subcore_barrier()
    pltpu.sync_copy(acc_sh.at[pl.ds(sid * 640, 640)],
                    out_hbm.at[cid, pl.ds(sid * 640, 640)])


def _matmul_body(x_ref, w_ref, z_ref):
    z_ref[...] = jnp.dot(x_ref[...], w_ref[...],
                         preferred_element_type=jnp.float32)


_matmul = pl.pallas_call(
    _matmul_body,
    out_shape=jax.ShapeDtypeStruct((N, D), jnp.float32),
)


def _scale_body(z_ref, dega_ref, degb_ref, g_ref, dinv_ref):
    dinv = lax.rsqrt(dega_ref[...] + degb_ref[...] + 1.0)
    dinv_ref[...] = dinv
    g_ref[...] = z_ref[...] * dinv


_scale = pl.pallas_call(
    _scale_body,
    out_shape=(jax.ShapeDtypeStruct((N, D), jnp.float32),
               jax.ShapeDtypeStruct((N, 1), jnp.float32)),
)


def _stage2_body(s1a_ref, s1b_ref, g1_ref, dinv_ref, b1_ref, wcat_ref, g2_ref):
    dinv = dinv_ref[...]
    h = jnp.maximum(
        dinv * (s1a_ref[...] + s1b_ref[...] + g1_ref[...]) + b1_ref[...], 0.0)
    g2_ref[...] = jnp.dot(h, wcat_ref[...],
                          preferred_element_type=jnp.float32) * dinv


_stage2 = pl.pallas_call(
    _stage2_body,
    out_shape=jax.ShapeDtypeStruct((N, D), jnp.float32),
)


def _stage3_body(s2a_ref, s2b_ref, g2_ref, dinv_ref, bcat_ref, out_ref):
    out_ref[...] = (dinv_ref[...]
                    * (s2a_ref[...] + s2b_ref[...] + g2_ref[...])
                    + bcat_ref[...])


_stage3 = pl.pallas_call(
    _stage3_body,
    out_shape=jax.ShapeDtypeStruct((N, D), jnp.float32),
)


def kernel(x, edge_index, W1, b1, W_mu, b_mu, W_ls, b_ls):
    src = edge_index[0]
    dst = edge_index[1]
    pad = E_PAD - E
    src_p = jnp.concatenate(
        [src, jnp.zeros((pad,), jnp.int32)]).reshape(NW * CPT, CHUNK)
    trash = N + jnp.arange(pad, dtype=jnp.int32) % (N_ACC - N)
    dst_p = jnp.concatenate([dst, trash]).reshape(NW * CPT, CHUNK)
    zeros1 = jnp.zeros((N_ACC,), jnp.float32)
    ones1 = jnp.ones((CHUNK,), jnp.float32)
    zeros2 = jnp.zeros((N_ACC, D), jnp.float32)

    degs = _deg_call(dst_p, zeros1, ones1).reshape(2, N_ACC)
    z1 = _matmul(x, W1)
    g1, dinv = _scale(z1, degs[0, :N, None], degs[1, :N, None])
    S1 = _edge_call(jnp.stack([g1, g1]), src_p, dst_p, zeros2)
    wcat = jnp.concatenate([W_mu, W_ls], axis=1)
    g2 = _stage2(S1[0, :N], S1[1, :N], g1, dinv, b1.reshape(1, D), wcat)
    S2 = _edge_call(jnp.stack([g2, g2]), src_p, dst_p, zeros2)
    bcat = jnp.concatenate([b_mu, b_ls]).reshape(1, D)
    out = _stage3(S2[0, :N], S2[1, :N], g2, dinv, bcat)
    return out[:, :64], out[:, 64:]


# speedup vs baseline: 1.2658x; 1.2658x over previous
import functools

import jax
import jax.numpy as jnp
from jax import lax
from jax.experimental import pallas as pl
from jax.experimental.pallas import tpu as pltpu
from jax.experimental.pallas import tpu_sc as plsc

N = 10000
D = 128
E = 320000
NW = 32
CHUNK = 128
CPT = 80
E_PAD = NW * CPT * CHUNK
N_ACC = 10240
GROUP = 16
K0 = 112
K1 = 2 * CPT - K0

_mesh = plsc.VectorSubcoreMesh(core_axis_name="c", subcore_axis_name="s")


_NP128 = N_ACC // 128


@functools.partial(
    pl.kernel,
    mesh=_mesh,
    out_type=jax.ShapeDtypeStruct((2 * N_ACC,), jnp.float32),
    scratch_types=[
        pltpu.VMEM((CPT, CHUNK), jnp.int32),
        pltpu.VMEM((CHUNK,), jnp.float32),
        pltpu.VMEM_SHARED((N_ACC,), jnp.float32),
    ],
)
def _deg_call(dst_hbm, zeros_hbm, ones_hbm, out_hbm, idx_v, ones_v, deg_sh):
    cid = lax.axis_index("c")
    sid = lax.axis_index("s")
    wid = cid * 16 + sid

    for k in range(5):
        p = sid + k * 16

        @pl.when(p < _NP128)
        def _():
            pltpu.sync_copy(zeros_hbm.at[pl.ds(p * 128, 128)],
                            deg_sh.at[pl.ds(p * 128, 128)])

    pltpu.sync_copy(ones_hbm, ones_v)
    pltpu.sync_copy(dst_hbm.at[pl.ds(wid * CPT, CPT)], idx_v)
    plsc.subcore_barrier()

    def chunk(j, carry):
        pltpu.sync_copy(ones_v, deg_sh.at[idx_v.at[j]], add=True)
        return carry

    lax.fori_loop(0, CPT, chunk, 0)
    plsc.subcore_barrier()
    for k in range(5):
        p = sid + k * 16

        @pl.when(p < _NP128)
        def _():
            pltpu.sync_copy(deg_sh.at[pl.ds(p * 128, 128)],
                            out_hbm.at[pl.ds(cid * N_ACC + p * 128, 128)])


@functools.partial(
    pl.kernel,
    mesh=_mesh,
    out_type=jax.ShapeDtypeStruct((2, N_ACC, D), jnp.float32),
    scratch_types=[
        pltpu.VMEM((GROUP, CHUNK), jnp.int32),
        pltpu.VMEM((GROUP, CHUNK), jnp.int32),
        pltpu.VMEM((CHUNK, D), jnp.float32),
        pltpu.VMEM((CHUNK, D), jnp.float32),
        pltpu.VMEM_SHARED((N_ACC, D), jnp.float32),
        pltpu.SemaphoreType.DMA,
        pltpu.SemaphoreType.DMA,
    ],
)
def _edge_call(g_hbm, src_hbm, dst_hbm, zeros_hbm, out_hbm,
               sidx_v, didx_v, rows0_v, rows1_v, acc_sh, sem0, sem1):
    cid = lax.axis_index("c")
    sid = lax.axis_index("s")
    pltpu.sync_copy(zeros_hbm.at[pl.ds(sid * 640, 640)],
                    acc_sh.at[pl.ds(sid * 640, 640)])
    plsc.subcore_barrier()

    ngroups = jnp.where(cid == 0, K0 // GROUP, K1 // GROUP)
    tile_base = sid * (2 * CPT) + jnp.where(cid == 0, 0, K0)

    def group(g, carry):
        base = pl.multiple_of(tile_base + g * GROUP, 8)
        pltpu.sync_copy(src_hbm.at[pl.ds(base, GROUP)], sidx_v)
        pltpu.sync_copy(dst_hbm.at[pl.ds(base, GROUP)], didx_v)
        pltpu.async_copy(g_hbm.at[cid].at[sidx_v.at[0]], rows0_v, sem0)

        def pair(t, carry2):
            j0 = 2 * t
            j1 = j0 + 1
            pltpu.async_copy(g_hbm.at[cid].at[sidx_v.at[j1]], rows1_v, sem1)
            pltpu.make_async_copy(g_hbm.at[cid].at[sidx_v.at[j0]], rows0_v,
                                  sem0).wait()
            pltpu.sync_copy(rows0_v, acc_sh.at[didx_v.at[j0]], add=True)

            @pl.when(j0 + 2 < GROUP)
            def _():
                pltpu.async_copy(g_hbm.at[cid].at[sidx_v.at[j0 + 2]], rows0_v, sem0)

            pltpu.make_async_copy(g_hbm.at[cid].at[sidx_v.at[j1]], rows1_v,
                                  sem1).wait()
            pltpu.sync_copy(rows1_v, acc_sh.at[didx_v.at[j1]], add=True)
            return carry2

        lax.fori_loop(0, GROUP // 2, pair, 0)
        return carry

    lax.fori_loop(0, ngroups, group, 0)
    plsc.subcore_barrier()
    pltpu.sync_copy(acc_sh.at[pl.ds(sid * 640, 640)],
                    out_hbm.at[cid, pl.ds(sid * 640, 640)])


def _matmul_body(x_ref, w_ref, z_ref):
    z_ref[...] = jnp.dot(x_ref[...], w_ref[...],
                         preferred_element_type=jnp.float32)


_matmul = pl.pallas_call(
    _matmul_body,
    out_shape=jax.ShapeDtypeStruct((N, D), jnp.float32),
)


def _scale_body(z_ref, dega_ref, degb_ref, g_ref, dinv_ref):
    dinv = lax.rsqrt(dega_ref[...] + degb_ref[...] + 1.0)
    dinv_ref[...] = dinv
    g_ref[...] = z_ref[...] * dinv


_scale = pl.pallas_call(
    _scale_body,
    out_shape=(jax.ShapeDtypeStruct((N, D), jnp.float32),
               jax.ShapeDtypeStruct((N, 1), jnp.float32)),
)


def _stage2_body(s1a_ref, s1b_ref, g1_ref, dinv_ref, b1_ref, wcat_ref, g2_ref):
    dinv = dinv_ref[...]
    h = jnp.maximum(
        dinv * (s1a_ref[...] + s1b_ref[...] + g1_ref[...]) + b1_ref[...], 0.0)
    g2_ref[...] = jnp.dot(h, wcat_ref[...],
                          preferred_element_type=jnp.float32) * dinv


_stage2 = pl.pallas_call(
    _stage2_body,
    out_shape=jax.ShapeDtypeStruct((N, D), jnp.float32),
)


def _stage3_body(s2a_ref, s2b_ref, g2_ref, dinv_ref, bcat_ref, out_ref):
    out_ref[...] = (dinv_ref[...]
                    * (s2a_ref[...] + s2b_ref[...] + g2_ref[...])
                    + bcat_ref[...])


_stage3 = pl.pallas_call(
    _stage3_body,
    out_shape=jax.ShapeDtypeStruct((N, D), jnp.float32),
)


def kernel(x, edge_index, W1, b1, W_mu, b_mu, W_ls, b_ls):
    src = edge_index[0]
    dst = edge_index[1]
    pad = E_PAD - E
    src_p = jnp.concatenate(
        [src, jnp.zeros((pad,), jnp.int32)]).reshape(NW * CPT, CHUNK)
    trash = N + jnp.arange(pad, dtype=jnp.int32) % (N_ACC - N)
    dst_p = jnp.concatenate([dst, trash]).reshape(NW * CPT, CHUNK)
    zeros1 = jnp.zeros((N_ACC,), jnp.float32)
    ones1 = jnp.ones((CHUNK,), jnp.float32)
    zeros2 = jnp.zeros((N_ACC, D), jnp.float32)

    degs = _deg_call(dst_p, zeros1, ones1).reshape(2, N_ACC)
    z1 = _matmul(x, W1)
    g1, dinv = _scale(z1, degs[0, :N, None], degs[1, :N, None])
    S1 = _edge_call(jnp.stack([g1, g1]), src_p, dst_p, zeros2)
    wcat = jnp.concatenate([W_mu, W_ls], axis=1)
    g2 = _stage2(S1[0, :N], S1[1, :N], g1, dinv, b1.reshape(1, D), wcat)
    S2 = _edge_call(jnp.stack([g2, g2]), src_p, dst_p, zeros2)
    bcat = jnp.concatenate([b_mu, b_ls]).reshape(1, D)
    out = _stage3(S2[0, :N], S2[1, :N], g2, dinv, bcat)
    return out[:, :64], out[:, 64:]


# speedup vs baseline: 1.2692x; 1.0028x over previous
import functools

import jax
import jax.numpy as jnp
from jax import lax
from jax.experimental import pallas as pl
from jax.experimental.pallas import tpu as pltpu
from jax.experimental.pallas import tpu_sc as plsc

N = 10000
D = 128
E = 320000
NW = 32
CHUNK = 128
CPT = 80
E_PAD = NW * CPT * CHUNK
N_ACC = 10240
GROUP = 32
K0 = 128
K1 = 2 * CPT - K0

_mesh = plsc.VectorSubcoreMesh(core_axis_name="c", subcore_axis_name="s")


_NP128 = N_ACC // 128


@functools.partial(
    pl.kernel,
    mesh=_mesh,
    out_type=jax.ShapeDtypeStruct((2 * N_ACC,), jnp.float32),
    scratch_types=[
        pltpu.VMEM((CPT, CHUNK), jnp.int32),
        pltpu.VMEM((CHUNK,), jnp.float32),
        pltpu.VMEM_SHARED((N_ACC,), jnp.float32),
    ],
)
def _deg_call(dst_hbm, zeros_hbm, ones_hbm, out_hbm, idx_v, ones_v, deg_sh):
    cid = lax.axis_index("c")
    sid = lax.axis_index("s")
    wid = cid * 16 + sid

    for k in range(5):
        p = sid + k * 16

        @pl.when(p < _NP128)
        def _():
            pltpu.sync_copy(zeros_hbm.at[pl.ds(p * 128, 128)],
                            deg_sh.at[pl.ds(p * 128, 128)])

    pltpu.sync_copy(ones_hbm, ones_v)
    pltpu.sync_copy(dst_hbm.at[pl.ds(wid * CPT, CPT)], idx_v)
    plsc.subcore_barrier()

    def chunk(j, carry):
        pltpu.sync_copy(ones_v, deg_sh.at[idx_v.at[j]], add=True)
        return carry

    lax.fori_loop(0, CPT, chunk, 0)
    plsc.subcore_barrier()
    for k in range(5):
        p = sid + k * 16

        @pl.when(p < _NP128)
        def _():
            pltpu.sync_copy(deg_sh.at[pl.ds(p * 128, 128)],
                            out_hbm.at[pl.ds(cid * N_ACC + p * 128, 128)])


@functools.partial(
    pl.kernel,
    mesh=_mesh,
    out_type=jax.ShapeDtypeStruct((2, N_ACC, D), jnp.float32),
    scratch_types=[
        pltpu.VMEM((GROUP, CHUNK), jnp.int32),
        pltpu.VMEM((GROUP, CHUNK), jnp.int32),
        pltpu.VMEM((CHUNK, D), jnp.float32),
        pltpu.VMEM((CHUNK, D), jnp.float32),
        pltpu.VMEM_SHARED((N_ACC, D), jnp.float32),
        pltpu.SemaphoreType.DMA,
        pltpu.SemaphoreType.DMA,
    ],
)
def _edge_call(g_hbm, src_hbm, dst_hbm, zeros_hbm, out_hbm,
               sidx_v, didx_v, rows0_v, rows1_v, acc_sh, sem0, sem1):
    cid = lax.axis_index("c")
    sid = lax.axis_index("s")
    pltpu.sync_copy(zeros_hbm.at[pl.ds(sid * 640, 640)],
                    acc_sh.at[pl.ds(sid * 640, 640)])
    plsc.subcore_barrier()

    ngroups = jnp.where(cid == 0, K0 // GROUP, K1 // GROUP)
    tile_base = sid * (2 * CPT) + jnp.where(cid == 0, 0, K0)

    def group(g, carry):
        base = pl.multiple_of(tile_base + g * GROUP, 8)
        pltpu.sync_copy(src_hbm.at[pl.ds(base, GROUP)], sidx_v)
        pltpu.sync_copy(dst_hbm.at[pl.ds(base, GROUP)], didx_v)
        pltpu.async_copy(g_hbm.at[cid].at[sidx_v.at[0]], rows0_v, sem0)

        def pair(t, carry2):
            j0 = 2 * t
            j1 = j0 + 1
            pltpu.async_copy(g_hbm.at[cid].at[sidx_v.at[j1]], rows1_v, sem1)
            pltpu.make_async_copy(g_hbm.at[cid].at[sidx_v.at[j0]], rows0_v,
                                  sem0).wait()
            pltpu.sync_copy(rows0_v, acc_sh.at[didx_v.at[j0]], add=True)

            @pl.when(j0 + 2 < GROUP)
            def _():
                pltpu.async_copy(g_hbm.at[cid].at[sidx_v.at[j0 + 2]], rows0_v, sem0)

            pltpu.make_async_copy(g_hbm.at[cid].at[sidx_v.at[j1]], rows1_v,
                                  sem1).wait()
            pltpu.sync_copy(rows1_v, acc_sh.at[didx_v.at[j1]], add=True)
            return carry2

        lax.fori_loop(0, GROUP // 2, pair, 0)
        return carry

    lax.fori_loop(0, ngroups, group, 0)
    plsc.subcore_barrier()
    pltpu.sync_copy(acc_sh.at[pl.ds(sid * 640, 640)],
                    out_hbm.at[cid, pl.ds(sid * 640, 640)])


def _matmul_body(x_ref, w_ref, z_ref):
    z_ref[...] = jnp.dot(x_ref[...], w_ref[...],
                         preferred_element_type=jnp.float32)


_matmul = pl.pallas_call(
    _matmul_body,
    out_shape=jax.ShapeDtypeStruct((N, D), jnp.float32),
)


def _scale_body(z_ref, dega_ref, degb_ref, g_ref, dinv_ref):
    dinv = lax.rsqrt(dega_ref[...] + degb_ref[...] + 1.0)
    dinv_ref[...] = dinv
    g = z_ref[...] * dinv
    g_ref[0] = g
    g_ref[1] = g


_scale = pl.pallas_call(
    _scale_body,
    out_shape=(jax.ShapeDtypeStruct((2, N, D), jnp.float32),
               jax.ShapeDtypeStruct((N, 1), jnp.float32)),
)


def _stage2_body(s1_ref, g1_ref, dinv_ref, b1_ref, wcat_ref, g2_ref):
    dinv = dinv_ref[...]
    h = jnp.maximum(
        dinv * (s1_ref[0, :N] + s1_ref[1, :N] + g1_ref[0]) + b1_ref[...], 0.0)
    g2 = jnp.dot(h, wcat_ref[...], preferred_element_type=jnp.float32) * dinv
    g2_ref[0] = g2
    g2_ref[1] = g2


_stage2 = pl.pallas_call(
    _stage2_body,
    out_shape=jax.ShapeDtypeStruct((2, N, D), jnp.float32),
)


def _stage3_body(s2_ref, g2_ref, dinv_ref, bcat_ref, mu_ref, ls_ref):
    out = (dinv_ref[...] * (s2_ref[0, :N] + s2_ref[1, :N] + g2_ref[0])
           + bcat_ref[...])
    mu_ref[...] = out[:, :64]
    ls_ref[...] = out[:, 64:]


_stage3 = pl.pallas_call(
    _stage3_body,
    out_shape=(jax.ShapeDtypeStruct((N, 64), jnp.float32),
               jax.ShapeDtypeStruct((N, 64), jnp.float32)),
)


def kernel(x, edge_index, W1, b1, W_mu, b_mu, W_ls, b_ls):
    src = edge_index[0]
    dst = edge_index[1]
    pad = E_PAD - E
    src_p = jnp.concatenate(
        [src, jnp.zeros((pad,), jnp.int32)]).reshape(NW * CPT, CHUNK)
    trash = N + jnp.arange(pad, dtype=jnp.int32) % (N_ACC - N)
    dst_p = jnp.concatenate([dst, trash]).reshape(NW * CPT, CHUNK)
    zeros1 = jnp.zeros((N_ACC,), jnp.float32)
    ones1 = jnp.ones((CHUNK,), jnp.float32)
    zeros2 = jnp.zeros((N_ACC, D), jnp.float32)

    degs = _deg_call(dst_p, zeros1, ones1).reshape(2, N_ACC)
    z1 = _matmul(x, W1)
    g1d, dinv = _scale(z1, degs[0, :N, None], degs[1, :N, None])
    S1 = _edge_call(g1d, src_p, dst_p, zeros2)
    wcat = jnp.concatenate([W_mu, W_ls], axis=1)
    g2d = _stage2(S1, g1d, dinv, b1.reshape(1, D), wcat)
    S2 = _edge_call(g2d, src_p, dst_p, zeros2)
    bcat = jnp.concatenate([b_mu, b_ls]).reshape(1, D)
    return _stage3(S2, g2d, dinv, bcat)


# speedup vs baseline: 1.2696x; 1.0003x over previous
import functools

import jax
import jax.numpy as jnp
from jax import lax
from jax.experimental import pallas as pl
from jax.experimental.pallas import tpu as pltpu
from jax.experimental.pallas import tpu_sc as plsc

N = 10000
D = 128
E = 320000
NW = 32
CHUNK = 128
CPT = 80
E_PAD = NW * CPT * CHUNK
N_ACC = 10240
GROUP = 32
K0 = 128
K1 = 2 * CPT - K0

_mesh = plsc.VectorSubcoreMesh(core_axis_name="c", subcore_axis_name="s")


_NP128 = N_ACC // 128


@functools.partial(
    pl.kernel,
    mesh=_mesh,
    out_type=jax.ShapeDtypeStruct((2 * N_ACC,), jnp.float32),
    scratch_types=[
        pltpu.VMEM((CPT, CHUNK), jnp.int32),
        pltpu.VMEM((CHUNK,), jnp.float32),
        pltpu.VMEM_SHARED((N_ACC,), jnp.float32),
    ],
)
def _deg_call(dst_hbm, zeros_hbm, ones_hbm, out_hbm, idx_v, ones_v, deg_sh):
    cid = lax.axis_index("c")
    sid = lax.axis_index("s")
    wid = cid * 16 + sid

    for k in range(5):
        p = sid + k * 16

        @pl.when(p < _NP128)
        def _():
            pltpu.sync_copy(zeros_hbm.at[pl.ds(p * 128, 128)],
                            deg_sh.at[pl.ds(p * 128, 128)])

    pltpu.sync_copy(ones_hbm, ones_v)
    pltpu.sync_copy(dst_hbm.at[pl.ds(wid * CPT, CPT)], idx_v)
    plsc.subcore_barrier()

    def chunk(j, carry):
        pltpu.sync_copy(ones_v, deg_sh.at[idx_v.at[j]], add=True)
        return carry

    lax.fori_loop(0, CPT, chunk, 0)
    plsc.subcore_barrier()
    for k in range(5):
        p = sid + k * 16

        @pl.when(p < _NP128)
        def _():
            pltpu.sync_copy(deg_sh.at[pl.ds(p * 128, 128)],
                            out_hbm.at[pl.ds(cid * N_ACC + p * 128, 128)])


@functools.partial(
    pl.kernel,
    mesh=_mesh,
    out_type=jax.ShapeDtypeStruct((2, N_ACC, D), jnp.float32),
    scratch_types=[
        pltpu.VMEM((GROUP, CHUNK), jnp.int32),
        pltpu.VMEM((GROUP, CHUNK), jnp.int32),
        pltpu.VMEM((CHUNK, D), jnp.float32),
        pltpu.VMEM((CHUNK, D), jnp.float32),
        pltpu.VMEM_SHARED((N_ACC, D), jnp.float32),
        pltpu.SemaphoreType.DMA,
        pltpu.SemaphoreType.DMA,
    ],
)
def _edge_call(g_hbm, src_hbm, dst_hbm, out_hbm,
               sidx_v, didx_v, rows0_v, rows1_v, acc_sh, sem0, sem1):
    cid = lax.axis_index("c")
    sid = lax.axis_index("s")

    def zrow(i, c):
        rows0_v[i // 8, pl.ds((i % 8) * 16, 16)] = jnp.zeros((16,), jnp.float32)
        return c

    lax.fori_loop(0, CHUNK * 8, zrow, 0)
    for k in range(5):
        pltpu.sync_copy(rows0_v, acc_sh.at[pl.ds(sid * 640 + k * 128, 128)])
    plsc.subcore_barrier()

    ngroups = jnp.where(cid == 0, K0 // GROUP, K1 // GROUP)
    tile_base = sid * (2 * CPT) + jnp.where(cid == 0, 0, K0)

    def group(g, carry):
        base = pl.multiple_of(tile_base + g * GROUP, 8)
        pltpu.sync_copy(src_hbm.at[pl.ds(base, GROUP)], sidx_v)
        pltpu.sync_copy(dst_hbm.at[pl.ds(base, GROUP)], didx_v)
        pltpu.async_copy(g_hbm.at[cid].at[sidx_v.at[0]], rows0_v, sem0)

        def pair(t, carry2):
            j0 = 2 * t
            j1 = j0 + 1
            pltpu.async_copy(g_hbm.at[cid].at[sidx_v.at[j1]], rows1_v, sem1)
            pltpu.make_async_copy(g_hbm.at[cid].at[sidx_v.at[j0]], rows0_v,
                                  sem0).wait()
            pltpu.sync_copy(rows0_v, acc_sh.at[didx_v.at[j0]], add=True)

            @pl.when(j0 + 2 < GROUP)
            def _():
                pltpu.async_copy(g_hbm.at[cid].at[sidx_v.at[j0 + 2]], rows0_v, sem0)

            pltpu.make_async_copy(g_hbm.at[cid].at[sidx_v.at[j1]], rows1_v,
                                  sem1).wait()
            pltpu.sync_copy(rows1_v, acc_sh.at[didx_v.at[j1]], add=True)
            return carry2

        lax.fori_loop(0, GROUP // 2, pair, 0)
        return carry

    lax.fori_loop(0, ngroups, group, 0)
    plsc.subcore_barrier()
    pltpu.sync_copy(acc_sh.at[pl.ds(sid * 640, 640)],
                    out_hbm.at[cid, pl.ds(sid * 640, 640)])


def _matmul_body(x_ref, w_ref, z_ref):
    z_ref[...] = jnp.dot(x_ref[...], w_ref[...],
                         preferred_element_type=jnp.float32)


_matmul = pl.pallas_call(
    _matmul_body,
    out_shape=jax.ShapeDtypeStruct((N, D), jnp.float32),
)


def _scale_body(z_ref, dega_ref, degb_ref, g_ref, dinv_ref):
    dinv = lax.rsqrt(dega_ref[...] + degb_ref[...] + 1.0)
    dinv_ref[...] = dinv
    g = z_ref[...] * dinv
    g_ref[0] = g
    g_ref[1] = g


_scale = pl.pallas_call(
    _scale_body,
    out_shape=(jax.ShapeDtypeStruct((2, N, D), jnp.float32),
               jax.ShapeDtypeStruct((N, 1), jnp.float32)),
)


def _stage2_body(s1_ref, g1_ref, dinv_ref, b1_ref, wcat_ref, g2_ref):
    dinv = dinv_ref[...]
    h = jnp.maximum(
        dinv * (s1_ref[0, :N] + s1_ref[1, :N] + g1_ref[0]) + b1_ref[...], 0.0)
    g2 = jnp.dot(h, wcat_ref[...], preferred_element_type=jnp.float32) * dinv
    g2_ref[0] = g2
    g2_ref[1] = g2


_stage2 = pl.pallas_call(
    _stage2_body,
    out_shape=jax.ShapeDtypeStruct((2, N, D), jnp.float32),
)


def _stage3_body(s2_ref, g2_ref, dinv_ref, bcat_ref, mu_ref, ls_ref):
    out = (dinv_ref[...] * (s2_ref[0, :N] + s2_ref[1, :N] + g2_ref[0])
           + bcat_ref[...])
    mu_ref[...] = out[:, :64]
    ls_ref[...] = out[:, 64:]


_stage3 = pl.pallas_call(
    _stage3_body,
    out_shape=(jax.ShapeDtypeStruct((N, 64), jnp.float32),
               jax.ShapeDtypeStruct((N, 64), jnp.float32)),
)


def kernel(x, edge_index, W1, b1, W_mu, b_mu, W_ls, b_ls):
    src = edge_index[0]
    dst = edge_index[1]
    pad = E_PAD - E
    src_p = jnp.concatenate(
        [src, jnp.zeros((pad,), jnp.int32)]).reshape(NW * CPT, CHUNK)
    trash = N + jnp.arange(pad, dtype=jnp.int32) % (N_ACC - N)
    dst_p = jnp.concatenate([dst, trash]).reshape(NW * CPT, CHUNK)
    zeros1 = jnp.zeros((N_ACC,), jnp.float32)
    ones1 = jnp.ones((CHUNK,), jnp.float32)

    degs = _deg_call(dst_p, zeros1, ones1).reshape(2, N_ACC)
    z1 = _matmul(x, W1)
    g1d, dinv = _scale(z1, degs[0, :N, None], degs[1, :N, None])
    S1 = _edge_call(g1d, src_p, dst_p)
    wcat = jnp.concatenate([W_mu, W_ls], axis=1)
    g2d = _stage2(S1, g1d, dinv, b1.reshape(1, D), wcat)
    S2 = _edge_call(g2d, src_p, dst_p)
    bcat = jnp.concatenate([b_mu, b_ls]).reshape(1, D)
    return _stage3(S2, g2d, dinv, bcat)
